# baseline (speedup 1.0000x reference)
import jax
import jax.numpy as jnp
from jax import lax
from jax.experimental import pallas as pl
from jax.experimental.pallas import tpu as pltpu
from jax.experimental.pallas import tpu_sc as plsc

_NC = 2
_NS = 16
_NW = _NC * _NS
_L = 16


def _build_sampler(B, d_obs, d_act):
    assert B % _NW == 0
    bpw = B // _NW
    assert bpw % 8 == 0 and bpw % _L == 0 and bpw <= 128
    mesh = plsc.VectorSubcoreMesh(core_axis_name="c", subcore_axis_name="s")

    def body(obs_hbm, act_hbm, rew_hbm, nobs_hbm, dw_hbm, idx_hbm,
             obs_out, act_out, rew_out, nobs_out, done_out,
             idx_v, idxw_v, obs_v, act_v, rew_v, nobs_v, dw_v, done_v, sem):
        wid = lax.axis_index("s") * _NC + lax.axis_index("c")
        base = wid * bpw
        pltpu.sync_copy(idx_hbm.at[pl.ds(base, bpw)], idx_v)
        for i in range(bpw // _L):
            s = pl.ds(i * _L, _L)
            idxw_v[s] = lax.shift_right_logical(idx_v[s], 2)
        copies = [
            pltpu.async_copy(obs_hbm.at[idx_v], obs_v, sem),
            pltpu.async_copy(act_hbm.at[idx_v], act_v, sem),
            pltpu.async_copy(rew_hbm.at[idx_v], rew_v, sem),
            pltpu.async_copy(nobs_hbm.at[idx_v], nobs_v, sem),
            pltpu.async_copy(dw_hbm.at[idxw_v], dw_v, sem),
        ]
        for cp in copies:
            cp.wait()
        for i in range(bpw // _L):
            s = pl.ds(i * _L, _L)
            sh = lax.shift_left(lax.bitwise_and(idx_v[s], 3), 3)
            done_v[s] = lax.bitwise_and(
                lax.shift_right_logical(dw_v[s], sh), 0xFF)
        dst = pl.ds(base, bpw)
        pltpu.sync_copy(obs_v, obs_out.at[dst])
        pltpu.sync_copy(act_v, act_out.at[dst])
        pltpu.sync_copy(rew_v, rew_out.at[dst])
        pltpu.sync_copy(nobs_v, nobs_out.at[dst])
        pltpu.sync_copy(done_v, done_out.at[dst])

    return pl.kernel(
        body,
        out_type=(
            jax.ShapeDtypeStruct((B, d_obs), jnp.float32),
            jax.ShapeDtypeStruct((B, d_act), jnp.float32),
            jax.ShapeDtypeStruct((B,), jnp.float32),
            jax.ShapeDtypeStruct((B, d_obs), jnp.float32),
            jax.ShapeDtypeStruct((B,), jnp.int32),
        ),
        mesh=mesh,
        compiler_params=pltpu.CompilerParams(use_tc_tiling_on_sc=False),
        scratch_types=[
            pltpu.VMEM((bpw,), jnp.int32),
            pltpu.VMEM((bpw,), jnp.int32),
            pltpu.VMEM((bpw, d_obs), jnp.float32),
            pltpu.VMEM((bpw, d_act), jnp.float32),
            pltpu.VMEM((bpw,), jnp.float32),
            pltpu.VMEM((bpw, d_obs), jnp.float32),
            pltpu.VMEM((bpw,), jnp.int32),
            pltpu.VMEM((bpw,), jnp.int32),
            pltpu.SemaphoreType.DMA,
        ],
    )


def kernel(obs_mem, action_mem, reward_mem, next_obs_mem, done_mem, idx):
    M, d_obs = obs_mem.shape
    d_act = action_mem.shape[1]
    B = idx.shape[0]
    done_words = lax.bitcast_convert_type(
        done_mem.astype(jnp.uint8).reshape(M // 4, 4), jnp.int32)
    sampler = _build_sampler(B, d_obs, d_act)
    obs_b, act_b, rew_b, nobs_b, done_i = sampler(
        obs_mem, action_mem, reward_mem, next_obs_mem, done_words, idx)
    return obs_b, act_b, rew_b, nobs_b, done_i.astype(jnp.bool_)
